# baseline (speedup 1.0000x reference)
import functools

import jax
import jax.numpy as jnp
from jax import lax
from jax.experimental import pallas as pl
from jax.experimental.pallas import tpu as pltpu
from jax.experimental.pallas import tpu_sc as plsc

_NUM_CORES = 2
_NUM_SUBCORES = 16
_NW = _NUM_CORES * _NUM_SUBCORES
_LG = 8
_BG = 4
_CHUNK = 128
_NBUF = 2
_LANES = 16
_SKEW = 133


@functools.partial(jax.jit, static_argnums=(3, 4, 5))
def _embed_gather(pairs, paroff, table2, bsz, lsz, d):
    l_per_w = lsz // _LG
    b_per_w = bsz // _BG
    chunks_per_l = b_per_w // _CHUNK
    nchunk = l_per_w * chunks_per_l
    assert nchunk % (2 * _NBUF) == 0

    mesh = plsc.VectorSubcoreMesh(
        core_axis_name="c", subcore_axis_name="s")

    @functools.partial(
        pl.kernel,
        out_type=jax.ShapeDtypeStruct((lsz, d, bsz), jnp.float32),
        mesh=mesh,
        scratch_types=[
            pltpu.VMEM((_CHUNK,), jnp.int32),
            pltpu.VMEM((_CHUNK,), jnp.int32),
            pltpu.VMEM((_CHUNK,), jnp.int32),
            pltpu.VMEM((_CHUNK,), jnp.int32),
            pltpu.VMEM((_CHUNK,), jnp.int32),
            pltpu.VMEM((_CHUNK,), jnp.int32),
            pltpu.VMEM((_CHUNK, 2 * 64), jnp.float32),
            pltpu.VMEM((_CHUNK, 2 * 64), jnp.float32),
            pltpu.VMEM((64, _LANES), jnp.int32),
            pltpu.VMEM((64, _CHUNK), jnp.float32),
            pltpu.VMEM((64, _CHUNK), jnp.float32),
            pltpu.SemaphoreType.DMA,
            pltpu.SemaphoreType.DMA,
            pltpu.SemaphoreType.DMA,
            pltpu.SemaphoreType.DMA,
            pltpu.SemaphoreType.DMA,
            pltpu.SemaphoreType.DMA,
            pltpu.SemaphoreType.DMA,
            pltpu.SemaphoreType.DMA,
            pltpu.SemaphoreType.DMA,
            pltpu.SemaphoreType.DMA,
        ],
        compiler_params=pltpu.CompilerParams(
            use_tc_tiling_on_sc=True, needs_layout_passes=False),
    )
    def gather_kernel(pairs_hbm, paroff_hbm, table_hbm, out_hbm,
                      i0, i1, p0, p1, p2, p3, r0, r1, diag, t0, t1,
                      is0, is1, ps0, ps1, ps2, ps3, g0, g1, s0, s1):
        wid = lax.axis_index("s") * _NUM_CORES + lax.axis_index("c")
        lg = wid // _BG
        bg = wid % _BG
        ibuf = [i0, i1]
        pbuf = [p0, p1, p2, p3]
        rows = [r0, r1]
        stage = [t0, t1]
        isem = [is0, is1]
        psem = [ps0, ps1, ps2, ps3]
        gsem = [g0, g1]
        ssem = [s0, s1]

        def flat_start(j):
            l = j // chunks_per_l
            tb = j % chunks_per_l
            return (lg * l_per_w + l) * bsz + bg * b_per_w + tb * _CHUNK

        def fetch_meta(j, b, pb):
            s = flat_start(j)
            pltpu.async_copy(
                pairs_hbm.at[pl.ds(s, _CHUNK)], ibuf[b], isem[b])
            pltpu.async_copy(
                paroff_hbm.at[pl.ds(s, _CHUNK)], pbuf[pb], psem[pb])

        def start_gather(j, b, pb):
            s = flat_start(j)
            pltpu.make_async_copy(
                pairs_hbm.at[pl.ds(s, _CHUNK)], ibuf[b], isem[b]).wait()
            pltpu.make_async_copy(
                paroff_hbm.at[pl.ds(s, _CHUNK)], pbuf[pb], psem[pb]).wait()
            pltpu.async_copy(table_hbm.at[ibuf[b]], rows[b], gsem[b])

        def out_dst(j):
            l = j // chunks_per_l
            tb = j % chunks_per_l
            return out_hbm.at[lg * l_per_w + l, :,
                              pl.ds(bg * b_per_w + tb * _CHUNK, _CHUNK)]

        iota = lax.iota(jnp.int32, _LANES)

        for c0 in range(64):
            diag[c0, pl.ds(0, _LANES)] = (iota + c0) & 63

        def transpose_select(b, pb):
            def body(grp, _):
                k0 = grp * _LANES
                row16 = iota + k0
                par16 = pbuf[pb][pl.ds(k0, _LANES)]
                for c0 in range(64):
                    cw = diag[c0, pl.ds(0, _LANES)]
                    v = plsc.load_gather(rows[b], [row16, par16 + cw])
                    plsc.store_scatter(stage[b], [cw, row16], v)
                return ()

            lax.fori_loop(0, _CHUNK // _LANES, body, (), unroll=False)

        for b in range(_NBUF):
            fetch_meta(b, b, b)
        for b in range(_NBUF):
            start_gather(b, b, b)

        def body(i, _):
            for b4 in range(2 * _NBUF):
                j = i + b4
                b = b4 % _NBUF
                pb_next = (b4 + _NBUF) % (2 * _NBUF)
                pltpu.make_async_copy(
                    table_hbm.at[ibuf[b]], rows[b], gsem[b]).wait()

                @pl.when(j + _NBUF < nchunk)
                def _():
                    fetch_meta(j + _NBUF, b, pb_next)

                @pl.when(j >= _NBUF)
                def _():
                    pltpu.make_async_copy(
                        stage[b], out_dst(j - _NBUF), ssem[b]).wait()

                transpose_select(b, b4)
                pltpu.async_copy(stage[b], out_dst(j), ssem[b])

                @pl.when(j + _NBUF < nchunk)
                def _():
                    start_gather(j + _NBUF, b, pb_next)
            return ()

        lax.fori_loop(0, nchunk // (2 * _NBUF),
                      lambda i, c: body(i * 2 * _NBUF, c), (),
                      unroll=False)

        for b in range(_NBUF):
            j = nchunk - _NBUF + b
            pltpu.make_async_copy(stage[b], out_dst(j), ssem[b]).wait()

    return gather_kernel(pairs, paroff, table2)


def kernel(x, table):
    bsz, lsz = x.shape
    v, d = table.shape
    xt = x.T.reshape(bsz * lsz).astype(jnp.int32)
    pairs = xt >> 1
    paroff = (xt & 1) << 6
    table2 = table.reshape(v // 2, 2 * d)
    out_t = _embed_gather(pairs, paroff, table2, bsz, lsz, d)
    return out_t.transpose(2, 0, 1)


# speedup vs baseline: 1.3390x; 1.3390x over previous
import functools

import jax
import jax.numpy as jnp
from jax import lax
from jax.experimental import pallas as pl
from jax.experimental.pallas import tpu as pltpu
from jax.experimental.pallas import tpu_sc as plsc

_NUM_CORES = 2
_NUM_SUBCORES = 16
_NW = _NUM_CORES * _NUM_SUBCORES
_CHUNK = 160
_RBUF = 4
_SBUF = 2
_LANES = 16


@functools.partial(jax.jit, static_argnums=(3, 4))
def _embed_gather(pairs, paroff, table2, n, d):
    per_w = n // _NW
    nchunk = per_w // _CHUNK
    assert nchunk % _RBUF == 0
    tiles = _CHUNK // 8

    mesh = plsc.VectorSubcoreMesh(
        core_axis_name="c", subcore_axis_name="s")

    @functools.partial(
        pl.kernel,
        out_type=jax.ShapeDtypeStruct((n // 8, 8, d), jnp.float32),
        mesh=mesh,
        scratch_types=(
            [pltpu.VMEM((_CHUNK,), jnp.int32) for _ in range(_RBUF)]
            + [pltpu.VMEM((_CHUNK,), jnp.int32) for _ in range(_RBUF)]
            + [pltpu.VMEM((_CHUNK, 2 * 64), jnp.float32)
               for _ in range(_RBUF)]
            + [pltpu.VMEM((tiles, 8, 64), jnp.float32)
               for _ in range(_SBUF)]
            + [pltpu.SemaphoreType.DMA for _ in range(3 * _RBUF + _SBUF)]
        ),
        compiler_params=pltpu.CompilerParams(use_tc_tiling_on_sc=True),
    )
    def gather_kernel(pairs_hbm, paroff_hbm, table_hbm, out_hbm, *bufs):
        ibuf = list(bufs[0:_RBUF])
        pbuf = list(bufs[_RBUF:2 * _RBUF])
        rows = list(bufs[2 * _RBUF:3 * _RBUF])
        stage = list(bufs[3 * _RBUF:3 * _RBUF + _SBUF])
        sems = bufs[3 * _RBUF + _SBUF:]
        isem = list(sems[0:_RBUF])
        psem = list(sems[_RBUF:2 * _RBUF])
        gsem = list(sems[2 * _RBUF:3 * _RBUF])
        ssem = list(sems[3 * _RBUF:3 * _RBUF + _SBUF])

        wid = lax.axis_index("s") * _NUM_CORES + lax.axis_index("c")
        base = wid * per_w

        def fetch_meta(j, rb):
            s = base + j * _CHUNK
            pltpu.async_copy(
                pairs_hbm.at[pl.ds(s, _CHUNK)], ibuf[rb], isem[rb])
            pltpu.async_copy(
                paroff_hbm.at[pl.ds(s, _CHUNK)], pbuf[rb], psem[rb])

        def start_gather(j, rb):
            s = base + j * _CHUNK
            pltpu.make_async_copy(
                pairs_hbm.at[pl.ds(s, _CHUNK)], ibuf[rb], isem[rb]).wait()
            pltpu.make_async_copy(
                paroff_hbm.at[pl.ds(s, _CHUNK)], pbuf[rb], psem[rb]).wait()
            pltpu.async_copy(table_hbm.at[ibuf[rb]], rows[rb], gsem[rb])

        def out_dst(j):
            return out_hbm.at[pl.ds((base + j * _CHUNK) // 8, tiles)]

        def select_half(rb, sb):
            def body(grp, _):
                k0 = grp * _LANES
                par16 = pbuf[rb][pl.ds(k0, _LANES)]
                for m in range(_LANES):
                    po = par16[m]
                    t = grp * 2 + m // 8
                    r = m % 8
                    for g in range(64 // _LANES):
                        v = rows[rb][k0 + m,
                                     pl.ds(po + g * _LANES, _LANES)]
                        stage[sb][t, r, pl.ds(g * _LANES, _LANES)] = v
                return ()

            lax.fori_loop(0, _CHUNK // _LANES, body, (), unroll=False)

        for rb in range(2):
            fetch_meta(rb, rb)
        for rb in range(2):
            start_gather(rb, rb)

        def body(i, _):
            for rb in range(_RBUF):
                j = i + rb
                sb = rb % _SBUF
                rb2 = (rb + 2) % _RBUF
                pltpu.make_async_copy(
                    table_hbm.at[ibuf[rb]], rows[rb], gsem[rb]).wait()

                @pl.when(j + 2 < nchunk)
                def _():
                    fetch_meta(j + 2, rb2)
                    start_gather(j + 2, rb2)

                @pl.when(j >= _SBUF)
                def _():
                    pltpu.make_async_copy(
                        stage[sb], out_dst(j - _SBUF), ssem[sb]).wait()

                select_half(rb, sb)
                pltpu.async_copy(stage[sb], out_dst(j), ssem[sb])
            return ()

        lax.fori_loop(0, nchunk // _RBUF,
                      lambda i, c: body(i * _RBUF, c), (),
                      unroll=False)

        for sb in range(_SBUF):
            j = nchunk - _SBUF + sb
            pltpu.make_async_copy(stage[sb], out_dst(j), ssem[sb]).wait()

    return gather_kernel(pairs, paroff, table2)


def kernel(x, table):
    bsz, lsz = x.shape
    v, d = table.shape
    n = bsz * lsz
    xf = x.reshape(n).astype(jnp.int32)
    pairs = xf >> 1
    paroff = (xf & 1) << 6
    table2 = table.reshape(v // 2, 2 * d)
    out3 = _embed_gather(pairs, paroff, table2, n, d)
    return out3.reshape(bsz, lsz, d)


# speedup vs baseline: 1.4407x; 1.0759x over previous
import functools

import jax
import jax.numpy as jnp
from jax import lax
from jax.experimental import pallas as pl
from jax.experimental.pallas import tpu as pltpu
from jax.experimental.pallas import tpu_sc as plsc

_NUM_CORES = 2
_NUM_SUBCORES = 16
_NW = _NUM_CORES * _NUM_SUBCORES
_CHUNK = 160
_RBUF = 4
_SBUF = 2
_LANES = 16


@functools.partial(jax.jit, static_argnums=(3, 4))
def _embed_gather(pairs, paroff, table2, n, d):
    per_w = n // _NW
    nchunk = per_w // _CHUNK
    assert nchunk % _RBUF == 0
    tiles = _CHUNK // 8

    mesh = plsc.VectorSubcoreMesh(
        core_axis_name="c", subcore_axis_name="s")

    @functools.partial(
        pl.kernel,
        out_type=jax.ShapeDtypeStruct((n // 8, 8, d), jnp.float32),
        mesh=mesh,
        scratch_types=(
            [pltpu.VMEM((_CHUNK,), jnp.int32) for _ in range(_RBUF)]
            + [pltpu.VMEM((_CHUNK,), jnp.int32) for _ in range(2 * _RBUF)]
            + [pltpu.VMEM((_CHUNK, 2 * 64), jnp.float32)
               for _ in range(_RBUF)]
            + [pltpu.VMEM((tiles, 8, 64), jnp.float32)
               for _ in range(_SBUF)]
            + [pltpu.SemaphoreType.DMA for _ in range(4 * _RBUF + _SBUF)]
        ),
        compiler_params=pltpu.CompilerParams(use_tc_tiling_on_sc=True),
    )
    def gather_kernel(pairs_hbm, paroff_hbm, table_hbm, out_hbm, *bufs):
        ibuf = list(bufs[0:_RBUF])
        pbuf = list(bufs[_RBUF:3 * _RBUF])
        rows = list(bufs[3 * _RBUF:4 * _RBUF])
        stage = list(bufs[4 * _RBUF:4 * _RBUF + _SBUF])
        sems = bufs[4 * _RBUF + _SBUF:]
        isem = list(sems[0:_RBUF])
        psem = list(sems[_RBUF:3 * _RBUF])
        gsem = list(sems[3 * _RBUF:4 * _RBUF])
        ssem = list(sems[4 * _RBUF:4 * _RBUF + _SBUF])

        wid = lax.axis_index("s") * _NUM_CORES + lax.axis_index("c")
        base = wid * per_w

        def fetch_meta(j, rb, pb):
            s = base + j * _CHUNK
            pltpu.async_copy(
                pairs_hbm.at[pl.ds(s, _CHUNK)], ibuf[rb], isem[rb])
            pltpu.async_copy(
                paroff_hbm.at[pl.ds(s, _CHUNK)], pbuf[pb], psem[pb])

        def start_gather(j, rb, pb):
            s = base + j * _CHUNK
            pltpu.make_async_copy(
                pairs_hbm.at[pl.ds(s, _CHUNK)], ibuf[rb], isem[rb]).wait()
            pltpu.make_async_copy(
                paroff_hbm.at[pl.ds(s, _CHUNK)], pbuf[pb], psem[pb]).wait()
            pltpu.async_copy(table_hbm.at[ibuf[rb]], rows[rb], gsem[rb])

        def out_dst(j):
            return out_hbm.at[pl.ds((base + j * _CHUNK) // 8, tiles)]

        def select_half(rb, pb, sb):
            def body(grp, _):
                k0 = grp * _LANES
                par16 = pbuf[pb][pl.ds(k0, _LANES)]
                for m in range(_LANES):
                    po = par16[m]
                    t = grp * 2 + m // 8
                    r = m % 8
                    for g in range(64 // _LANES):
                        v = rows[rb][k0 + m,
                                     pl.ds(po + g * _LANES, _LANES)]
                        stage[sb][t, r, pl.ds(g * _LANES, _LANES)] = v
                return ()

            lax.fori_loop(0, _CHUNK // _LANES, body, (), unroll=False)

        for j0 in range(_RBUF):
            fetch_meta(j0, j0, j0)
        for j0 in range(2):
            start_gather(j0, j0, j0)

        def body(i, _):
            for r8 in range(2 * _RBUF):
                j = i + r8
                rb = r8 % _RBUF
                sb = r8 % _SBUF
                pltpu.make_async_copy(
                    table_hbm.at[ibuf[rb]], rows[rb], gsem[rb]).wait()

                @pl.when(j + _RBUF < nchunk)
                def _():
                    fetch_meta(j + _RBUF, rb, (r8 + _RBUF) % (2 * _RBUF))

                @pl.when(j + 2 < nchunk)
                def _():
                    start_gather(j + 2, (rb + 2) % _RBUF,
                                 (r8 + 2) % (2 * _RBUF))

                @pl.when(j >= _SBUF)
                def _():
                    pltpu.make_async_copy(
                        stage[sb], out_dst(j - _SBUF), ssem[sb]).wait()

                select_half(rb, r8, sb)
                pltpu.async_copy(stage[sb], out_dst(j), ssem[sb])
            return ()

        lax.fori_loop(0, nchunk // (2 * _RBUF),
                      lambda i, c: body(i * 2 * _RBUF, c), (),
                      unroll=False)

        for sb in range(_SBUF):
            j = nchunk - _SBUF + sb
            pltpu.make_async_copy(stage[sb], out_dst(j), ssem[sb]).wait()

    return gather_kernel(pairs, paroff, table2)


def kernel(x, table):
    bsz, lsz = x.shape
    v, d = table.shape
    n = bsz * lsz
    xf = x.reshape(n).astype(jnp.int32)
    pairs = xf >> 1
    paroff = (xf & 1) << 6
    table2 = table.reshape(v // 2, 2 * d)
    out3 = _embed_gather(pairs, paroff, table2, n, d)
    return out3.reshape(bsz, lsz, d)
